# baseline (speedup 1.0000x reference)
import functools

import jax
import jax.numpy as jnp
from jax import lax
from jax.experimental import pallas as pl
from jax.experimental.pallas import tpu as pltpu
from jax.experimental.pallas import tpu_sc as plsc

N_NODES = 10000
N_EDGES = 320000
NUM_ELEMENTS = 100
HIDDEN = 128
NUM_RADIAL = 16

EDGE_BLOCK = 16000

NC = 2
NS = 16
NW = NC * NS
ROWS_W = 320
LAST_W = N_NODES - (NW - 1) * ROWS_W
GATHER = 128


def _chunks(total):
    off, out = 0, []
    while off < total:
        n = min(GATHER, total - off)
        out.append((off, n))
        off += n
    return out


def _proj_kernel(rb_ref, w_ref, b_ref, s_ref, t_ref, m_ref):
    rb = rb_ref[...]
    for k, o_ref in enumerate((s_ref, t_ref, m_ref)):
        y = jnp.dot(rb, w_ref[..., k * HIDDEN:(k + 1) * HIDDEN],
                    preferred_element_type=jnp.float32)
        o_ref[...] = y + b_ref[..., k * HIDDEN:(k + 1) * HIDDEN]


@functools.partial(
    pl.kernel,
    mesh=plsc.VectorSubcoreMesh(core_axis_name="c", subcore_axis_name="s"),
    out_type=jax.ShapeDtypeStruct((N_NODES, HIDDEN), jnp.float32),
    scratch_types=[
        pltpu.VMEM((ROWS_W,), jnp.int32),
        pltpu.VMEM((ROWS_W, HIDDEN), jnp.float32),
        pltpu.SemaphoreType.DMA,
    ],
)
def _sc_gather(idx_hbm, table_hbm, out_hbm, idx_v, rows_v, sem):
    wid = lax.axis_index("s") * NC + lax.axis_index("c")
    base = wid * ROWS_W

    def _run(nrows):
        pltpu.sync_copy(idx_hbm.at[pl.ds(base, nrows)],
                        idx_v.at[pl.ds(0, nrows)])
        handles = [
            pltpu.async_copy(table_hbm.at[idx_v.at[pl.ds(off, n)]],
                             rows_v.at[pl.ds(off, n)], sem)
            for off, n in _chunks(nrows)
        ]
        for hnd in handles:
            hnd.wait()
        pltpu.sync_copy(rows_v.at[pl.ds(0, nrows)],
                        out_hbm.at[pl.ds(base, nrows)])

    @pl.when(wid < NW - 1)
    def _():
        _run(ROWS_W)

    @pl.when(wid == NW - 1)
    def _():
        _run(LAST_W)


def kernel(atomic_numbers, radial_basis, emb_table, W, b):
    idx = atomic_numbers.astype(jnp.int32) - 1
    h = _sc_gather(idx, emb_table)

    rb16 = radial_basis.astype(jnp.bfloat16)
    w16 = W.astype(jnp.bfloat16)
    b2 = b.reshape(1, HIDDEN * 3)
    grid_e = N_EDGES // EDGE_BLOCK
    out_block = pl.BlockSpec((EDGE_BLOCK, HIDDEN), lambda i: (i, 0))
    s, t, m = pl.pallas_call(
        _proj_kernel,
        grid=(grid_e,),
        in_specs=[
            pl.BlockSpec((EDGE_BLOCK, NUM_RADIAL), lambda i: (i, 0)),
            pl.BlockSpec((NUM_RADIAL, HIDDEN * 3), lambda i: (0, 0)),
            pl.BlockSpec((1, HIDDEN * 3), lambda i: (0, 0)),
        ],
        out_specs=[out_block, out_block, out_block],
        out_shape=[jax.ShapeDtypeStruct((N_EDGES, HIDDEN), jnp.float32)] * 3,
        compiler_params=pltpu.CompilerParams(
            dimension_semantics=("parallel",),
            vmem_limit_bytes=100 * 1024 * 1024),
    )(rb16, w16, b2)

    return (h, m, s, t)


# speedup vs baseline: 1.0086x; 1.0086x over previous
import functools

import jax
import jax.numpy as jnp
from jax import lax
from jax.experimental import pallas as pl
from jax.experimental.pallas import tpu as pltpu
from jax.experimental.pallas import tpu_sc as plsc

N_NODES = 10000
N_EDGES = 320000
NUM_ELEMENTS = 100
HIDDEN = 128
NUM_RADIAL = 16

EDGE_BLOCK = 16000

NC = 2
NS = 16
NW = NC * NS
CHUNK = 80
TOTAL_CHUNKS = N_NODES // CHUNK
MAX_ROUNDS = -(-TOTAL_CHUNKS // NW)


def _proj_kernel(rb_ref, w_ref, b_ref, s_ref, t_ref, m_ref):
    rb = rb_ref[...]
    for k, o_ref in enumerate((s_ref, t_ref, m_ref)):
        y = jnp.dot(rb, w_ref[..., k * HIDDEN:(k + 1) * HIDDEN],
                    preferred_element_type=jnp.float32)
        o_ref[...] = y + b_ref[..., k * HIDDEN:(k + 1) * HIDDEN]


@functools.partial(
    pl.kernel,
    mesh=plsc.VectorSubcoreMesh(core_axis_name="c", subcore_axis_name="s"),
    out_type=jax.ShapeDtypeStruct((N_NODES, HIDDEN), jnp.float32),
    scratch_types=[
        pltpu.VMEM((CHUNK,), jnp.int32),
        pltpu.VMEM((CHUNK, HIDDEN), jnp.float32),
        pltpu.SemaphoreType.DMA,
    ],
)
def _sc_gather(idx_hbm, table_hbm, out_hbm, idx_v, rows_v, sem):
    wid = lax.axis_index("s") * NC + lax.axis_index("c")
    for r in range(MAX_ROUNDS):
        c = wid + r * NW

        @pl.when(c < TOTAL_CHUNKS)
        def _():
            base = c * CHUNK
            pltpu.sync_copy(idx_hbm.at[pl.ds(base, CHUNK)], idx_v)
            pltpu.async_copy(table_hbm.at[idx_v], rows_v, sem).wait()
            pltpu.sync_copy(rows_v, out_hbm.at[pl.ds(base, CHUNK)])


def kernel(atomic_numbers, radial_basis, emb_table, W, b):
    idx = atomic_numbers.astype(jnp.int32) - 1
    h = _sc_gather(idx, emb_table)

    rb16 = radial_basis.astype(jnp.bfloat16)
    w16 = W.astype(jnp.bfloat16)
    b2 = b.reshape(1, HIDDEN * 3)
    grid_e = N_EDGES // EDGE_BLOCK
    out_block = pl.BlockSpec((EDGE_BLOCK, HIDDEN), lambda i: (i, 0))
    s, t, m = pl.pallas_call(
        _proj_kernel,
        grid=(grid_e,),
        in_specs=[
            pl.BlockSpec((EDGE_BLOCK, NUM_RADIAL), lambda i: (i, 0)),
            pl.BlockSpec((NUM_RADIAL, HIDDEN * 3), lambda i: (0, 0)),
            pl.BlockSpec((1, HIDDEN * 3), lambda i: (0, 0)),
        ],
        out_specs=[out_block, out_block, out_block],
        out_shape=[jax.ShapeDtypeStruct((N_EDGES, HIDDEN), jnp.float32)] * 3,
        compiler_params=pltpu.CompilerParams(
            vmem_limit_bytes=100 * 1024 * 1024),
    )(rb16, w16, b2)

    return (h, m, s, t)
